# Initial kernel scaffold; baseline (speedup 1.0000x reference)
import jax
import jax.numpy as jnp
from jax.experimental import pallas as pl


def kernel(adj):
    raise NotImplementedError("write your pallas kernel here")


# speedup vs baseline: 8.9376x; 8.9376x over previous
import jax
import jax.numpy as jnp
from jax import lax
from jax.experimental import pallas as pl
from jax.experimental.pallas import tpu as pltpu


def _fw_body(a_ref, o_ref):
    n = a_ref.shape[-1]
    a = a_ref[0]
    rows = lax.broadcasted_iota(jnp.int32, (n, n), 0)
    cols = lax.broadcasted_iota(jnp.int32, (n, n), 1)
    eye = rows == cols
    w = jnp.where((a != 0.0) | eye, a, jnp.inf)
    d0 = jnp.where(eye, 0.0, w)

    o_ref[0] = d0

    def step(k, _):
        d = o_ref[0]
        row = o_ref[0, pl.ds(k, 1), :]
        col = pltpu.roll(d, -k, axis=1)[:, 0:1]
        o_ref[0] = jnp.minimum(d, col + row)
        return 0

    lax.fori_loop(0, n, step, 0)


def kernel(adj):
    n = adj.shape[-1]
    batch = adj.shape[0] * adj.shape[1]
    a = adj.reshape(batch, n, n)
    out = pl.pallas_call(
        _fw_body,
        grid=(batch,),
        in_specs=[pl.BlockSpec((1, n, n), lambda b: (b, 0, 0))],
        out_specs=pl.BlockSpec((1, n, n), lambda b: (b, 0, 0)),
        out_shape=jax.ShapeDtypeStruct((batch, n, n), adj.dtype),
    )(a)
    return out.reshape(adj.shape)


# speedup vs baseline: 14.6400x; 1.6380x over previous
import jax
import jax.numpy as jnp
from jax import lax
from jax.experimental import pallas as pl
from jax.experimental.pallas import tpu as pltpu


def _fw_body(a_ref, o_ref):
    n = a_ref.shape[-1]
    a = a_ref[0]
    rows = lax.broadcasted_iota(jnp.int32, (n, n), 0)
    cols = lax.broadcasted_iota(jnp.int32, (n, n), 1)
    eye = rows == cols
    w = jnp.where((a != 0.0) | eye, a, jnp.inf)
    d0 = jnp.where(eye, 0.0, w)

    o_ref[0] = d0

    B = 8

    def block(kb, _):
        base = kb * B
        r = o_ref[0, pl.ds(base, B), :]
        g = pltpu.roll(r, -base, axis=1)[:, 0:B]
        for t in range(B):
            g = jnp.minimum(g, g[:, t : t + 1] + g[t : t + 1, :])
        for t in range(B):
            r = jnp.minimum(r, g[:, t : t + 1] + r[t : t + 1, :])
        d = o_ref[0]
        c0 = pltpu.roll(d, -base, axis=1)[:, 0:B]
        for t in range(B):
            d = jnp.minimum(d, c0[:, t : t + 1] + r[t : t + 1, :])
        o_ref[0] = d
        return 0

    lax.fori_loop(0, n // B, block, 0)


def kernel(adj):
    n = adj.shape[-1]
    batch = adj.shape[0] * adj.shape[1]
    a = adj.reshape(batch, n, n)
    out = pl.pallas_call(
        _fw_body,
        grid=(batch,),
        in_specs=[pl.BlockSpec((1, n, n), lambda b: (b, 0, 0))],
        out_specs=pl.BlockSpec((1, n, n), lambda b: (b, 0, 0)),
        out_shape=jax.ShapeDtypeStruct((batch, n, n), adj.dtype),
    )(a)
    return out.reshape(adj.shape)
